# baseline (speedup 1.0000x reference)
import functools

import jax
import jax.numpy as jnp
from jax import lax
from jax.experimental import pallas as pl
from jax.experimental.pallas import tpu as pltpu
from jax.experimental.pallas import tpu_sc as plsc

N = 10000
NP = 10240
E = 320000
NC, NS = 2, 16
NW = NC * NS
EB = 80
NB = 128
EW = EB * NB
EP = NW * EW
NB0 = 128
NB1 = 256 - NB0
RPT = NP // NS


@functools.cache
def _get_mesh():
    return plsc.VectorSubcoreMesh(core_axis_name="c", subcore_axis_name="s",
                                  num_cores=NC, num_subcores=NS)


@functools.cache
def _get_deg_kernel():
    @functools.partial(
        pl.kernel,
        out_type=jax.ShapeDtypeStruct((NC, NP), jnp.float32),
        mesh=_get_mesh(),
        scratch_types=[
            pltpu.VMEM((NB, EB), jnp.int32),
            pltpu.VMEM((EB,), jnp.float32),
            pltpu.VMEM((RPT,), jnp.float32),
            pltpu.VMEM_SHARED((NP,), jnp.float32),
        ],
    )
    def _deg_kernel(dst_hbm, out_hbm, dst_v, ones_v, zero_v, acc_sh):
        cid = lax.axis_index("c")
        sid = lax.axis_index("s")
        wid = cid * NS + sid

        for i in range(EB // 16):
            ones_v[pl.ds(i * 16, 16)] = jnp.ones((16,), jnp.float32)
        for i in range(RPT // 16):
            zero_v[pl.ds(i * 16, 16)] = jnp.zeros((16,), jnp.float32)
        pltpu.sync_copy(zero_v, acc_sh.at[pl.ds(sid * RPT, RPT)])
        pltpu.sync_copy(dst_hbm.at[wid], dst_v)
        plsc.subcore_barrier()

        def body(i, carry):
            pltpu.sync_copy(ones_v, acc_sh.at[dst_v.at[i]], add=True)
            return carry

        lax.fori_loop(0, NB, body, 0)
        plsc.subcore_barrier()
        pltpu.sync_copy(acc_sh.at[pl.ds(sid * RPT, RPT)],
                        out_hbm.at[cid, pl.ds(sid * RPT, RPT)])

    return _deg_kernel


@functools.cache
def _make_agg(D):
    @functools.partial(
        pl.kernel,
        out_type=jax.ShapeDtypeStruct((NC, NP, D), jnp.float32),
        mesh=_get_mesh(),
        scratch_types=[
            pltpu.VMEM((NB0 * EB,), jnp.int32),
            pltpu.VMEM((2, EB), jnp.int32),
            pltpu.VMEM((2, EB), jnp.int32),
            pltpu.VMEM((2, EB, D), jnp.float32),
            pltpu.VMEM_SHARED((NP, D), jnp.float32),
            pltpu.SemaphoreType.DMA,
            pltpu.SemaphoreType.DMA,
        ],
    )
    def agg(y_hbm, packed_hbm, zeros_hbm, out_hbm, packed_v, src_v, dst_v,
            rows_v, acc_sh, gs0, gs1):
        cid = lax.axis_index("c")
        sid = lax.axis_index("s")
        gsem = (gs0, gs1)

        def unpack(i, b):
            for c in range(EB // 16):
                p = packed_v[pl.ds(i * EB + c * 16, 16)]
                src_v[b, pl.ds(c * 16, 16)] = jnp.bitwise_and(p, 16383)
                dst_v[b, pl.ds(c * 16, 16)] = jnp.right_shift(p, 14)

        def start_gather(rb):
            pltpu.async_copy(y_hbm.at[src_v.at[rb]], rows_v.at[rb], gsem[rb])

        def wait_gather(rb):
            pltpu.make_async_copy(y_hbm.at[pl.ds(0, EB)], rows_v.at[rb],
                                  gsem[rb]).wait()

        def scatter(rb):
            pltpu.sync_copy(rows_v.at[rb], acc_sh.at[dst_v.at[rb]], add=True)

        pltpu.sync_copy(zeros_hbm.at[pl.ds(sid * RPT, RPT)],
                        acc_sh.at[pl.ds(sid * RPT, RPT)])
        plsc.subcore_barrier()

        def run(base, nbatch):
            pltpu.sync_copy(packed_hbm.at[pl.ds(base, nbatch * EB)],
                            packed_v.at[pl.ds(0, nbatch * EB)])
            for rb in range(2):
                unpack(rb, rb)
                start_gather(rb)

            def body(q, carry):
                i0 = q * 2
                for rb in range(2):
                    i = i0 + rb
                    wait_gather(rb)
                    scatter(rb)

                    @pl.when(i + 2 < nbatch)
                    def _():
                        unpack(i + 2, rb)
                        start_gather(rb)
                return carry

            lax.fori_loop(0, nbatch // 2, body, 0)

        @pl.when(cid == 0)
        def _sc0():
            run(sid * NB0 * EB, NB0)

        @pl.when(cid == 1)
        def _sc1():
            run(NS * NB0 * EB + sid * NB1 * EB, NB1)

        plsc.subcore_barrier()
        pltpu.sync_copy(acc_sh.at[pl.ds(sid * RPT, RPT)],
                        out_hbm.at[cid, pl.ds(sid * RPT, RPT)])

    return agg


_BLK = 1024
_G = NP // _BLK


def _dinv_body(deg_ref, out_ref):
    deg = deg_ref[0:1, :] + deg_ref[1:2, :] + 1.0
    out_ref[...] = lax.rsqrt(deg)


def _dinv_kernel(deg_partials):
    return pl.pallas_call(
        _dinv_body,
        out_shape=jax.ShapeDtypeStruct((1, NP), jnp.float32),
    )(deg_partials)


def _scale_mm_body(x_ref, w_ref, dinv_ref, out_ref):
    xw = jnp.dot(x_ref[...], w_ref[...], preferred_element_type=jnp.float32)
    out_ref[...] = dinv_ref[...] * xw


def _scale_mm(x, w, dinv):
    m, k = x.shape
    n = w.shape[1]
    return pl.pallas_call(
        _scale_mm_body,
        grid=(_G,),
        in_specs=[
            pl.BlockSpec((_BLK, k), lambda i: (i, 0)),
            pl.BlockSpec((k, n), lambda i: (0, 0)),
            pl.BlockSpec((_BLK, 1), lambda i: (i, 0)),
        ],
        out_specs=pl.BlockSpec((_BLK, n), lambda i: (i, 0)),
        out_shape=jax.ShapeDtypeStruct((m, n), jnp.float32),
    )(x, w, dinv)


def _layer_body(a_ref, y_ref, dinv_ref, b_ref, w_ref, out_ref):
    s = a_ref[0] + a_ref[1] + y_ref[...]
    h = jnp.maximum(dinv_ref[...] * s + b_ref[...], 0.0)
    out_ref[...] = dinv_ref[...] * jnp.dot(
        h, w_ref[...], preferred_element_type=jnp.float32)


def _layer(agg, y, dinv, b, w):
    d = y.shape[1]
    n = w.shape[1]
    return pl.pallas_call(
        _layer_body,
        grid=(_G,),
        in_specs=[
            pl.BlockSpec((NC, _BLK, d), lambda i: (0, i, 0)),
            pl.BlockSpec((_BLK, d), lambda i: (i, 0)),
            pl.BlockSpec((_BLK, 1), lambda i: (i, 0)),
            pl.BlockSpec((1, d), lambda i: (0, 0)),
            pl.BlockSpec((d, n), lambda i: (0, 0)),
        ],
        out_specs=pl.BlockSpec((_BLK, n), lambda i: (i, 0)),
        out_shape=jax.ShapeDtypeStruct((NP, n), jnp.float32),
    )(agg, y, dinv, b, w)


def _final_body(a_ref, y_ref, dinv_ref, b_ref, m1_ref, mb1_ref, m2_ref,
                mb2_ref, m3_ref, mb3_ref, out_ref):
    s = a_ref[0] + a_ref[1] + y_ref[...]
    h = jnp.maximum(dinv_ref[...] * s + b_ref[...], 0.0)
    h = jnp.maximum(
        jnp.dot(h, m1_ref[...], preferred_element_type=jnp.float32)
        + mb1_ref[...], 0.0)
    h = jnp.maximum(
        jnp.dot(h, m2_ref[...], preferred_element_type=jnp.float32)
        + mb2_ref[...], 0.0)
    out_ref[...] = jnp.dot(
        h, m3_ref[...], preferred_element_type=jnp.float32) + mb3_ref[...]


def _final(agg, y, dinv, b, m1, mb1, m2, mb2, m3, mb3):
    d = y.shape[1]
    return pl.pallas_call(
        _final_body,
        grid=(_G,),
        in_specs=[
            pl.BlockSpec((NC, _BLK, d), lambda i: (0, i, 0)),
            pl.BlockSpec((_BLK, d), lambda i: (i, 0)),
            pl.BlockSpec((_BLK, 1), lambda i: (i, 0)),
            pl.BlockSpec((1, d), lambda i: (0, 0)),
            pl.BlockSpec((d, 64), lambda i: (0, 0)),
            pl.BlockSpec((1, 64), lambda i: (0, 0)),
            pl.BlockSpec((64, 64), lambda i: (0, 0)),
            pl.BlockSpec((1, 64), lambda i: (0, 0)),
            pl.BlockSpec((64, 128), lambda i: (0, 0)),
            pl.BlockSpec((1, 128), lambda i: (0, 0)),
        ],
        out_specs=pl.BlockSpec((_BLK, 128), lambda i: (i, 0)),
        out_shape=jax.ShapeDtypeStruct((NP, 128), jnp.float32),
    )(agg, y, dinv, b, m1, mb1, m2, mb2, m3, mb3)


def kernel(x, edge_index, W1, b1, W2, b2, W3, b3, W4, b4,
           M1, mb1, M2, mb2, M3, mb3):
    ei = edge_index.astype(jnp.int32)
    pad_iota = jnp.arange(EP - E, dtype=jnp.int32)
    srcf = jnp.concatenate([ei[0], pad_iota % N])
    dstf = jnp.concatenate([ei[1], N + pad_iota % (NP - N)])
    packed = jnp.bitwise_or(srcf, jnp.left_shift(dstf, 14))
    dst = dstf.reshape(NW, NB, EB)

    xp = jnp.pad(x, ((0, NP - N), (0, 0)))
    W4p = jnp.pad(W4, ((0, 0), (0, 128 - W4.shape[1])))
    b4p = jnp.pad(b4, (0, 128 - b4.shape[0])).reshape(1, 128)
    M1p = jnp.pad(M1, ((0, 128 - M1.shape[0]), (0, 0)))
    M3p = jnp.pad(M3, ((0, 0), (0, 128 - M3.shape[1])))
    mb3p = jnp.pad(mb3, (0, 128 - mb3.shape[0])).reshape(1, 128)
    b1r, b2r, b3r = b1.reshape(1, -1), b2.reshape(1, -1), b3.reshape(1, -1)
    mb1r, mb2r = mb1.reshape(1, -1), mb2.reshape(1, -1)

    _agg128 = _make_agg(128)
    zeros = jnp.zeros((NP, 128), jnp.float32)

    degp = _get_deg_kernel()(dst)
    dinv = _dinv_kernel(degp).reshape(NP, 1)

    y1 = _scale_mm(xp, W1, dinv)
    a1 = _agg128(y1, packed, zeros)
    y2 = _layer(a1, y1, dinv, b1r, W2)
    a2 = _agg128(y2, packed, zeros)
    y3 = _layer(a2, y2, dinv, b2r, W3)
    a3 = _agg128(y3, packed, zeros)
    y4 = _layer(a3, y3, dinv, b3r, W4p)
    a4 = _agg128(y4, packed, zeros)
    out = _final(a4, y4, dinv, b4p, M1p, mb1r, M2, mb2r, M3p, mb3p)
    return out[:N, :M3.shape[1]]


# speedup vs baseline: 1.1032x; 1.1032x over previous
import functools

import jax
import jax.numpy as jnp
from jax import lax
from jax.experimental import pallas as pl
from jax.experimental.pallas import tpu as pltpu
from jax.experimental.pallas import tpu_sc as plsc

N = 10000
NP = 10240
E = 320000
NC, NS = 2, 16
NW = NC * NS
EB = 128
NB = 80
EW = EB * NB
EP = NW * EW
RPT = NP // NS


@functools.cache
def _get_mesh():
    return plsc.VectorSubcoreMesh(core_axis_name="c", subcore_axis_name="s",
                                  num_cores=NC, num_subcores=NS)


@functools.cache
def _get_deg_kernel():
    @functools.partial(
        pl.kernel,
        out_type=jax.ShapeDtypeStruct((NC, NP), jnp.float32),
        mesh=_get_mesh(),
        scratch_types=[
            pltpu.VMEM((NB, EB), jnp.int32),
            pltpu.VMEM((EB,), jnp.float32),
            pltpu.VMEM((RPT,), jnp.float32),
            pltpu.VMEM_SHARED((NP,), jnp.float32),
        ],
    )
    def _deg_kernel(dst_hbm, out_hbm, dst_v, ones_v, zero_v, acc_sh):
        cid = lax.axis_index("c")
        sid = lax.axis_index("s")
        wid = cid * NS + sid

        for i in range(EB // 16):
            ones_v[pl.ds(i * 16, 16)] = jnp.ones((16,), jnp.float32)
        for i in range(RPT // 16):
            zero_v[pl.ds(i * 16, 16)] = jnp.zeros((16,), jnp.float32)
        pltpu.sync_copy(zero_v, acc_sh.at[pl.ds(sid * RPT, RPT)])
        pltpu.sync_copy(dst_hbm.at[wid], dst_v)
        plsc.subcore_barrier()

        def body(i, carry):
            pltpu.sync_copy(ones_v, acc_sh.at[dst_v.at[i]], add=True)
            return carry

        lax.fori_loop(0, NB, body, 0)
        plsc.subcore_barrier()
        pltpu.sync_copy(acc_sh.at[pl.ds(sid * RPT, RPT)],
                        out_hbm.at[cid, pl.ds(sid * RPT, RPT)])

    return _deg_kernel


@functools.cache
def _make_agg(D):
    @functools.partial(
        pl.kernel,
        out_type=jax.ShapeDtypeStruct((NC, NP, D), jnp.float32),
        mesh=_get_mesh(),
        scratch_types=[
            pltpu.VMEM((4, EB), jnp.int32),
            pltpu.VMEM((4, EB), jnp.int32),
            pltpu.VMEM((2, EB, D), jnp.float32),
            pltpu.VMEM_SHARED((NP, D), jnp.float32),
            pltpu.SemaphoreType.DMA,
            pltpu.SemaphoreType.DMA,
            pltpu.SemaphoreType.DMA,
            pltpu.SemaphoreType.DMA,
            pltpu.SemaphoreType.DMA,
            pltpu.SemaphoreType.DMA,
        ],
    )
    def agg(y_hbm, src_hbm, dst_hbm, zeros_hbm, out_hbm, src_v, dst_v,
            rows_v, acc_sh, gs0, gs1, is0, is1, is2, is3):
        cid = lax.axis_index("c")
        sid = lax.axis_index("s")
        wid = cid * NS + sid
        gsem = (gs0, gs1)
        isem = (is0, is1, is2, is3)

        def load_idx(i, b):
            pltpu.async_copy(src_hbm.at[wid, i], src_v.at[b], isem[b])
            pltpu.async_copy(dst_hbm.at[wid, i], dst_v.at[b], isem[b])

        def wait_idx(b):
            pltpu.make_async_copy(src_hbm.at[0, 0], src_v.at[b],
                                  isem[b]).wait()
            pltpu.make_async_copy(dst_hbm.at[0, 0], dst_v.at[b],
                                  isem[b]).wait()

        def start_gather(b4, rb):
            pltpu.async_copy(y_hbm.at[src_v.at[b4]], rows_v.at[rb], gsem[rb])

        def wait_gather(rb):
            pltpu.make_async_copy(y_hbm.at[pl.ds(0, EB)], rows_v.at[rb],
                                  gsem[rb]).wait()

        pltpu.sync_copy(zeros_hbm.at[pl.ds(sid * RPT, RPT)],
                        acc_sh.at[pl.ds(sid * RPT, RPT)])
        plsc.subcore_barrier()

        for b in range(4):
            load_idx(b, b)
        for rb in range(2):
            wait_idx(rb)
            start_gather(rb, rb)

        def body(q, carry):
            i0 = q * 4
            for b in range(4):
                i = i0 + b
                rb = b % 2
                wait_gather(rb)
                pltpu.sync_copy(rows_v.at[rb], acc_sh.at[dst_v.at[b]],
                                add=True)

                @pl.when(i + 4 < NB)
                def _():
                    load_idx(i + 4, b)

                @pl.when(i + 2 < NB)
                def _():
                    wait_idx((b + 2) % 4)
                    start_gather((b + 2) % 4, rb)
            return carry

        lax.fori_loop(0, NB // 4, body, 0)
        plsc.subcore_barrier()
        pltpu.sync_copy(acc_sh.at[pl.ds(sid * RPT, RPT)],
                        out_hbm.at[cid, pl.ds(sid * RPT, RPT)])

    return agg


_BLK = 1024
_G = NP // _BLK


def _dinv_body(deg_ref, out_ref):
    deg = deg_ref[0:1, :] + deg_ref[1:2, :] + 1.0
    out_ref[...] = lax.rsqrt(deg)


def _dinv_kernel(deg_partials):
    return pl.pallas_call(
        _dinv_body,
        out_shape=jax.ShapeDtypeStruct((1, NP), jnp.float32),
    )(deg_partials)


def _scale_mm_body(x_ref, w_ref, dinv_ref, out_ref):
    xw = jnp.dot(x_ref[...], w_ref[...], preferred_element_type=jnp.float32)
    out_ref[...] = dinv_ref[...] * xw


def _scale_mm(x, w, dinv):
    m, k = x.shape
    n = w.shape[1]
    return pl.pallas_call(
        _scale_mm_body,
        grid=(_G,),
        in_specs=[
            pl.BlockSpec((_BLK, k), lambda i: (i, 0)),
            pl.BlockSpec((k, n), lambda i: (0, 0)),
            pl.BlockSpec((_BLK, 1), lambda i: (i, 0)),
        ],
        out_specs=pl.BlockSpec((_BLK, n), lambda i: (i, 0)),
        out_shape=jax.ShapeDtypeStruct((m, n), jnp.float32),
    )(x, w, dinv)


def _layer_body(a_ref, y_ref, dinv_ref, b_ref, w_ref, out_ref):
    s = a_ref[0] + a_ref[1] + y_ref[...]
    h = jnp.maximum(dinv_ref[...] * s + b_ref[...], 0.0)
    out_ref[...] = dinv_ref[...] * jnp.dot(
        h, w_ref[...], preferred_element_type=jnp.float32)


def _layer(agg, y, dinv, b, w):
    d = y.shape[1]
    n = w.shape[1]
    return pl.pallas_call(
        _layer_body,
        grid=(_G,),
        in_specs=[
            pl.BlockSpec((NC, _BLK, d), lambda i: (0, i, 0)),
            pl.BlockSpec((_BLK, d), lambda i: (i, 0)),
            pl.BlockSpec((_BLK, 1), lambda i: (i, 0)),
            pl.BlockSpec((1, d), lambda i: (0, 0)),
            pl.BlockSpec((d, n), lambda i: (0, 0)),
        ],
        out_specs=pl.BlockSpec((_BLK, n), lambda i: (i, 0)),
        out_shape=jax.ShapeDtypeStruct((NP, n), jnp.float32),
    )(agg, y, dinv, b, w)


def _final_body(a_ref, y_ref, dinv_ref, b_ref, m1_ref, mb1_ref, m2_ref,
                mb2_ref, m3_ref, mb3_ref, out_ref):
    s = a_ref[0] + a_ref[1] + y_ref[...]
    h = jnp.maximum(dinv_ref[...] * s + b_ref[...], 0.0)
    h = jnp.maximum(
        jnp.dot(h, m1_ref[...], preferred_element_type=jnp.float32)
        + mb1_ref[...], 0.0)
    h = jnp.maximum(
        jnp.dot(h, m2_ref[...], preferred_element_type=jnp.float32)
        + mb2_ref[...], 0.0)
    out_ref[...] = jnp.dot(
        h, m3_ref[...], preferred_element_type=jnp.float32) + mb3_ref[...]


def _final(agg, y, dinv, b, m1, mb1, m2, mb2, m3, mb3):
    d = y.shape[1]
    return pl.pallas_call(
        _final_body,
        grid=(_G,),
        in_specs=[
            pl.BlockSpec((NC, _BLK, d), lambda i: (0, i, 0)),
            pl.BlockSpec((_BLK, d), lambda i: (i, 0)),
            pl.BlockSpec((_BLK, 1), lambda i: (i, 0)),
            pl.BlockSpec((1, d), lambda i: (0, 0)),
            pl.BlockSpec((d, 64), lambda i: (0, 0)),
            pl.BlockSpec((1, 64), lambda i: (0, 0)),
            pl.BlockSpec((64, 64), lambda i: (0, 0)),
            pl.BlockSpec((1, 64), lambda i: (0, 0)),
            pl.BlockSpec((64, 128), lambda i: (0, 0)),
            pl.BlockSpec((1, 128), lambda i: (0, 0)),
        ],
        out_specs=pl.BlockSpec((_BLK, 128), lambda i: (i, 0)),
        out_shape=jax.ShapeDtypeStruct((NP, 128), jnp.float32),
    )(agg, y, dinv, b, m1, mb1, m2, mb2, m3, mb3)


def kernel(x, edge_index, W1, b1, W2, b2, W3, b3, W4, b4,
           M1, mb1, M2, mb2, M3, mb3):
    ei = edge_index.astype(jnp.int32)
    pad_iota = jnp.arange(EP - E, dtype=jnp.int32)
    srcf = jnp.concatenate([ei[0], pad_iota % N])
    dstf = jnp.concatenate([ei[1], N + pad_iota % (NP - N)])
    src = srcf.reshape(NW, NB, EB)
    dst = dstf.reshape(NW, NB, EB)

    xp = jnp.pad(x, ((0, NP - N), (0, 0)))
    W4p = jnp.pad(W4, ((0, 0), (0, 128 - W4.shape[1])))
    b4p = jnp.pad(b4, (0, 128 - b4.shape[0])).reshape(1, 128)
    M1p = jnp.pad(M1, ((0, 128 - M1.shape[0]), (0, 0)))
    M3p = jnp.pad(M3, ((0, 0), (0, 128 - M3.shape[1])))
    mb3p = jnp.pad(mb3, (0, 128 - mb3.shape[0])).reshape(1, 128)
    b1r, b2r, b3r = b1.reshape(1, -1), b2.reshape(1, -1), b3.reshape(1, -1)
    mb1r, mb2r = mb1.reshape(1, -1), mb2.reshape(1, -1)

    _agg128 = _make_agg(128)
    zeros = jnp.zeros((NP, 128), jnp.float32)

    degp = _get_deg_kernel()(dst)
    dinv = _dinv_kernel(degp).reshape(NP, 1)

    y1 = _scale_mm(xp, W1, dinv)
    a1 = _agg128(y1, src, dst, zeros)
    y2 = _layer(a1, y1, dinv, b1r, W2)
    a2 = _agg128(y2, src, dst, zeros)
    y3 = _layer(a2, y2, dinv, b2r, W3)
    a3 = _agg128(y3, src, dst, zeros)
    y4 = _layer(a3, y3, dinv, b3r, W4p)
    a4 = _agg128(y4, src, dst, zeros)
    out = _final(a4, y4, dinv, b4p, M1p, mb1r, M2, mb2r, M3p, mb3p)
    return out[:N, :M3.shape[1]]


# speedup vs baseline: 1.1226x; 1.0176x over previous
import functools

import jax
import jax.numpy as jnp
from jax import lax
from jax.experimental import pallas as pl
from jax.experimental.pallas import tpu as pltpu
from jax.experimental.pallas import tpu_sc as plsc

N = 10000
NP = 10240
E = 320000
NC, NS = 2, 16
NW = NC * NS
EB = 128
NB = 80
EW = EB * NB
EP = NW * EW
RPT = NP // NS


@functools.cache
def _get_mesh():
    return plsc.VectorSubcoreMesh(core_axis_name="c", subcore_axis_name="s",
                                  num_cores=NC, num_subcores=NS)


@functools.cache
def _get_deg_kernel():
    @functools.partial(
        pl.kernel,
        out_type=jax.ShapeDtypeStruct((NC, NP), jnp.float32),
        mesh=_get_mesh(),
        scratch_types=[
            pltpu.VMEM((NB, EB), jnp.int32),
            pltpu.VMEM((EB,), jnp.float32),
            pltpu.VMEM((RPT,), jnp.float32),
            pltpu.VMEM_SHARED((NP,), jnp.float32),
        ],
    )
    def _deg_kernel(dst_hbm, out_hbm, dst_v, ones_v, zero_v, acc_sh):
        cid = lax.axis_index("c")
        sid = lax.axis_index("s")
        wid = cid * NS + sid

        for i in range(EB // 16):
            ones_v[pl.ds(i * 16, 16)] = jnp.ones((16,), jnp.float32)
        for i in range(RPT // 16):
            zero_v[pl.ds(i * 16, 16)] = jnp.zeros((16,), jnp.float32)
        pltpu.sync_copy(zero_v, acc_sh.at[pl.ds(sid * RPT, RPT)])
        pltpu.sync_copy(dst_hbm.at[wid], dst_v)
        plsc.subcore_barrier()

        def body(i, carry):
            pltpu.sync_copy(ones_v, acc_sh.at[dst_v.at[i]], add=True)
            return carry

        lax.fori_loop(0, NB, body, 0)
        plsc.subcore_barrier()
        pltpu.sync_copy(acc_sh.at[pl.ds(sid * RPT, RPT)],
                        out_hbm.at[cid, pl.ds(sid * RPT, RPT)])

    return _deg_kernel


@functools.cache
def _make_agg(D):
    @functools.partial(
        pl.kernel,
        out_type=jax.ShapeDtypeStruct((NC, NP, D), jnp.float32),
        mesh=_get_mesh(),
        scratch_types=[
            pltpu.VMEM((4, EB), jnp.int32),
            pltpu.VMEM((4, EB), jnp.int32),
            pltpu.VMEM((2, EB, D), jnp.float32),
            pltpu.VMEM_SHARED((NP, D), jnp.float32),
            pltpu.SemaphoreType.DMA,
            pltpu.SemaphoreType.DMA,
            pltpu.SemaphoreType.DMA,
            pltpu.SemaphoreType.DMA,
            pltpu.SemaphoreType.DMA,
            pltpu.SemaphoreType.DMA,
        ],
    )
    def agg(y_hbm, src_hbm, dst_hbm, zeros_hbm, out_hbm, src_v, dst_v,
            rows_v, acc_sh, gs0, gs1, is0, is1, is2, is3):
        cid = lax.axis_index("c")
        sid = lax.axis_index("s")
        wid = cid * NS + sid
        gsem = (gs0, gs1)
        isem = (is0, is1, is2, is3)

        def load_idx(i, b):
            pltpu.async_copy(src_hbm.at[wid, i], src_v.at[b], isem[b])
            pltpu.async_copy(dst_hbm.at[wid, i], dst_v.at[b], isem[b])

        def wait_idx(b):
            pltpu.make_async_copy(src_hbm.at[0, 0], src_v.at[b],
                                  isem[b]).wait()
            pltpu.make_async_copy(dst_hbm.at[0, 0], dst_v.at[b],
                                  isem[b]).wait()

        def start_gather(b4, rb):
            pltpu.async_copy(y_hbm.at[src_v.at[b4]], rows_v.at[rb], gsem[rb])

        def wait_gather(rb):
            pltpu.make_async_copy(y_hbm.at[pl.ds(0, EB)], rows_v.at[rb],
                                  gsem[rb]).wait()

        for b in range(4):
            load_idx(b, b)
        for rb in range(2):
            wait_idx(rb)
            start_gather(rb, rb)

        pltpu.sync_copy(zeros_hbm.at[pl.ds(sid * RPT, RPT)],
                        acc_sh.at[pl.ds(sid * RPT, RPT)])
        plsc.subcore_barrier()

        def body(q, carry):
            i0 = q * 4
            for b in range(4):
                i = i0 + b
                rb = b % 2
                wait_gather(rb)
                pltpu.sync_copy(rows_v.at[rb], acc_sh.at[dst_v.at[b]],
                                add=True)

                @pl.when(i + 4 < NB)
                def _():
                    load_idx(i + 4, b)

                @pl.when(i + 2 < NB)
                def _():
                    wait_idx((b + 2) % 4)
                    start_gather((b + 2) % 4, rb)
            return carry

        lax.fori_loop(0, NB // 4, body, 0)
        plsc.subcore_barrier()
        pltpu.sync_copy(acc_sh.at[pl.ds(sid * RPT, RPT)],
                        out_hbm.at[cid, pl.ds(sid * RPT, RPT)])

    return agg


_BLK = 1024
_G = NP // _BLK


def _dinv_body(deg_ref, out_ref):
    deg = deg_ref[0:1, :] + deg_ref[1:2, :] + 1.0
    out_ref[...] = lax.rsqrt(deg)


def _dinv_kernel(deg_partials):
    return pl.pallas_call(
        _dinv_body,
        out_shape=jax.ShapeDtypeStruct((1, NP), jnp.float32),
    )(deg_partials)


def _scale_mm_body(x_ref, w_ref, dinv_ref, out_ref):
    xw = jnp.dot(x_ref[...], w_ref[...], preferred_element_type=jnp.float32)
    out_ref[...] = dinv_ref[...] * xw


def _scale_mm(x, w, dinv):
    m, k = x.shape
    n = w.shape[1]
    return pl.pallas_call(
        _scale_mm_body,
        grid=(_G,),
        in_specs=[
            pl.BlockSpec((_BLK, k), lambda i: (i, 0)),
            pl.BlockSpec((k, n), lambda i: (0, 0)),
            pl.BlockSpec((_BLK, 1), lambda i: (i, 0)),
        ],
        out_specs=pl.BlockSpec((_BLK, n), lambda i: (i, 0)),
        out_shape=jax.ShapeDtypeStruct((m, n), jnp.float32),
    )(x, w, dinv)


def _layer_body(a_ref, y_ref, dinv_ref, b_ref, w_ref, out_ref):
    s = a_ref[0] + a_ref[1] + y_ref[...]
    h = jnp.maximum(dinv_ref[...] * s + b_ref[...], 0.0)
    out_ref[...] = dinv_ref[...] * jnp.dot(
        h, w_ref[...], preferred_element_type=jnp.float32)


def _layer(agg, y, dinv, b, w):
    d = y.shape[1]
    n = w.shape[1]
    return pl.pallas_call(
        _layer_body,
        grid=(_G,),
        in_specs=[
            pl.BlockSpec((NC, _BLK, d), lambda i: (0, i, 0)),
            pl.BlockSpec((_BLK, d), lambda i: (i, 0)),
            pl.BlockSpec((_BLK, 1), lambda i: (i, 0)),
            pl.BlockSpec((1, d), lambda i: (0, 0)),
            pl.BlockSpec((d, n), lambda i: (0, 0)),
        ],
        out_specs=pl.BlockSpec((_BLK, n), lambda i: (i, 0)),
        out_shape=jax.ShapeDtypeStruct((NP, n), jnp.float32),
    )(agg, y, dinv, b, w)


def _final_body(a_ref, y_ref, dinv_ref, b_ref, m1_ref, mb1_ref, m2_ref,
                mb2_ref, m3_ref, mb3_ref, out_ref):
    s = a_ref[0] + a_ref[1] + y_ref[...]
    h = jnp.maximum(dinv_ref[...] * s + b_ref[...], 0.0)
    h = jnp.maximum(
        jnp.dot(h, m1_ref[...], preferred_element_type=jnp.float32)
        + mb1_ref[...], 0.0)
    h = jnp.maximum(
        jnp.dot(h, m2_ref[...], preferred_element_type=jnp.float32)
        + mb2_ref[...], 0.0)
    out_ref[...] = jnp.dot(
        h, m3_ref[...], preferred_element_type=jnp.float32) + mb3_ref[...]


def _final(agg, y, dinv, b, m1, mb1, m2, mb2, m3, mb3):
    d = y.shape[1]
    return pl.pallas_call(
        _final_body,
        grid=(_G,),
        in_specs=[
            pl.BlockSpec((NC, _BLK, d), lambda i: (0, i, 0)),
            pl.BlockSpec((_BLK, d), lambda i: (i, 0)),
            pl.BlockSpec((_BLK, 1), lambda i: (i, 0)),
            pl.BlockSpec((1, d), lambda i: (0, 0)),
            pl.BlockSpec((d, 64), lambda i: (0, 0)),
            pl.BlockSpec((1, 64), lambda i: (0, 0)),
            pl.BlockSpec((64, 64), lambda i: (0, 0)),
            pl.BlockSpec((1, 64), lambda i: (0, 0)),
            pl.BlockSpec((64, 128), lambda i: (0, 0)),
            pl.BlockSpec((1, 128), lambda i: (0, 0)),
        ],
        out_specs=pl.BlockSpec((_BLK, 128), lambda i: (i, 0)),
        out_shape=jax.ShapeDtypeStruct((NP, 128), jnp.float32),
    )(agg, y, dinv, b, m1, mb1, m2, mb2, m3, mb3)


def kernel(x, edge_index, W1, b1, W2, b2, W3, b3, W4, b4,
           M1, mb1, M2, mb2, M3, mb3):
    ei = edge_index.astype(jnp.int32)
    pad_iota = jnp.arange(EP - E, dtype=jnp.int32)
    srcf = jnp.concatenate([ei[0], pad_iota % N])
    dstf = jnp.concatenate([ei[1], N + pad_iota % (NP - N)])
    src = srcf.reshape(NW, NB, EB)
    dst = dstf.reshape(NW, NB, EB)

    xp = jnp.pad(x, ((0, NP - N), (0, 0)))
    W4p = jnp.pad(W4, ((0, 0), (0, 128 - W4.shape[1])))
    b4p = jnp.pad(b4, (0, 128 - b4.shape[0])).reshape(1, 128)
    M1p = jnp.pad(M1, ((0, 128 - M1.shape[0]), (0, 0)))
    M3p = jnp.pad(M3, ((0, 0), (0, 128 - M3.shape[1])))
    mb3p = jnp.pad(mb3, (0, 128 - mb3.shape[0])).reshape(1, 128)
    b1r, b2r, b3r = b1.reshape(1, -1), b2.reshape(1, -1), b3.reshape(1, -1)
    mb1r, mb2r = mb1.reshape(1, -1), mb2.reshape(1, -1)

    _agg128 = _make_agg(128)
    zeros = jnp.zeros((NP, 128), jnp.float32)

    degp = _get_deg_kernel()(dst)
    dinv = _dinv_kernel(degp).reshape(NP, 1)

    y1 = _scale_mm(xp, W1, dinv)
    a1 = _agg128(y1, src, dst, zeros)
    y2 = _layer(a1, y1, dinv, b1r, W2)
    a2 = _agg128(y2, src, dst, zeros)
    y3 = _layer(a2, y2, dinv, b2r, W3)
    a3 = _agg128(y3, src, dst, zeros)
    y4 = _layer(a3, y3, dinv, b3r, W4p)
    a4 = _agg128(y4, src, dst, zeros)
    out = _final(a4, y4, dinv, b4p, M1p, mb1r, M2, mb2r, M3p, mb3p)
    return out[:N, :M3.shape[1]]


# speedup vs baseline: 1.1238x; 1.0011x over previous
import functools

import jax
import jax.numpy as jnp
from jax import lax
from jax.experimental import pallas as pl
from jax.experimental.pallas import tpu as pltpu
from jax.experimental.pallas import tpu_sc as plsc

N = 10000
NP = 10240
E = 320000
NC, NS = 2, 16
NW = NC * NS
EB = 128
NB = 80
EW = EB * NB
EP = NW * EW
RPT = NP // NS


@functools.cache
def _get_mesh():
    return plsc.VectorSubcoreMesh(core_axis_name="c", subcore_axis_name="s",
                                  num_cores=NC, num_subcores=NS)


@functools.cache
def _get_deg_kernel():
    @functools.partial(
        pl.kernel,
        out_type=jax.ShapeDtypeStruct((NC, NP), jnp.float32),
        mesh=_get_mesh(),
        scratch_types=[
            pltpu.VMEM((NB, EB), jnp.int32),
            pltpu.VMEM((EB,), jnp.float32),
            pltpu.VMEM((RPT,), jnp.float32),
            pltpu.VMEM_SHARED((NP,), jnp.float32),
        ],
    )
    def _deg_kernel(dst_hbm, out_hbm, dst_v, ones_v, zero_v, acc_sh):
        cid = lax.axis_index("c")
        sid = lax.axis_index("s")
        wid = cid * NS + sid

        for i in range(EB // 16):
            ones_v[pl.ds(i * 16, 16)] = jnp.ones((16,), jnp.float32)
        for i in range(RPT // 16):
            zero_v[pl.ds(i * 16, 16)] = jnp.zeros((16,), jnp.float32)
        pltpu.sync_copy(zero_v, acc_sh.at[pl.ds(sid * RPT, RPT)])
        pltpu.sync_copy(dst_hbm.at[wid], dst_v)
        plsc.subcore_barrier()

        def body(i, carry):
            pltpu.sync_copy(ones_v, acc_sh.at[dst_v.at[i]], add=True)
            return carry

        lax.fori_loop(0, NB, body, 0)
        plsc.subcore_barrier()
        pltpu.sync_copy(acc_sh.at[pl.ds(sid * RPT, RPT)],
                        out_hbm.at[cid, pl.ds(sid * RPT, RPT)])

    return _deg_kernel


@functools.cache
def _make_agg(D):
    @functools.partial(
        pl.kernel,
        out_type=jax.ShapeDtypeStruct((NC, NP, D), jnp.float32),
        mesh=_get_mesh(),
        scratch_types=[
            pltpu.VMEM((4, EB), jnp.int32),
            pltpu.VMEM((4, EB), jnp.int32),
            pltpu.VMEM((2, EB, D), jnp.float32),
            pltpu.VMEM_SHARED((NP, D), jnp.float32),
            pltpu.SemaphoreType.DMA,
            pltpu.SemaphoreType.DMA,
            pltpu.SemaphoreType.DMA,
            pltpu.SemaphoreType.DMA,
            pltpu.SemaphoreType.DMA,
            pltpu.SemaphoreType.DMA,
        ],
    )
    def agg(y_hbm, src_hbm, dst_hbm, zeros_hbm, out_hbm, src_v, dst_v,
            rows_v, acc_sh, gs0, gs1, is0, is1, is2, is3):
        cid = lax.axis_index("c")
        sid = lax.axis_index("s")
        wid = cid * NS + sid
        gsem = (gs0, gs1)
        isem = (is0, is1, is2, is3)

        def load_idx(i, b):
            pltpu.async_copy(src_hbm.at[wid, i], src_v.at[b], isem[b])
            pltpu.async_copy(dst_hbm.at[wid, i], dst_v.at[b], isem[b])

        def wait_idx(b):
            pltpu.make_async_copy(src_hbm.at[0, 0], src_v.at[b],
                                  isem[b]).wait()
            pltpu.make_async_copy(dst_hbm.at[0, 0], dst_v.at[b],
                                  isem[b]).wait()

        def start_gather(b4, rb):
            pltpu.async_copy(y_hbm.at[src_v.at[b4]], rows_v.at[rb], gsem[rb])

        def wait_gather(rb):
            pltpu.make_async_copy(y_hbm.at[pl.ds(0, EB)], rows_v.at[rb],
                                  gsem[rb]).wait()

        for b in range(4):
            load_idx(b, b)
        for rb in range(2):
            wait_idx(rb)
            start_gather(rb, rb)

        pltpu.sync_copy(zeros_hbm.at[pl.ds(sid * RPT, RPT)],
                        acc_sh.at[pl.ds(sid * RPT, RPT)])
        plsc.subcore_barrier()

        def body(q, carry):
            i0 = q * 4
            for b in range(4):
                i = i0 + b
                rb = b % 2
                wait_gather(rb)
                pltpu.sync_copy(rows_v.at[rb], acc_sh.at[dst_v.at[b]],
                                add=True)

                @pl.when(i + 4 < NB)
                def _():
                    load_idx(i + 4, b)

                @pl.when(i + 2 < NB)
                def _():
                    wait_idx((b + 2) % 4)
                    start_gather((b + 2) % 4, rb)
            return carry

        lax.fori_loop(0, NB // 4, body, 0)
        plsc.subcore_barrier()
        pltpu.sync_copy(acc_sh.at[pl.ds(sid * RPT, RPT)],
                        out_hbm.at[cid, pl.ds(sid * RPT, RPT)])

    return agg


_BLK = 1024
_G = NP // _BLK


def _dinv_body(deg_ref, out_ref):
    deg = deg_ref[0:1, :] + deg_ref[1:2, :] + 1.0
    out_ref[...] = lax.rsqrt(deg)


def _dinv_kernel(deg_partials):
    return pl.pallas_call(
        _dinv_body,
        out_shape=jax.ShapeDtypeStruct((1, NP), jnp.float32),
    )(deg_partials)


def _scale_mm_body(x_ref, w_ref, dinv_ref, out_ref):
    xw = jnp.dot(x_ref[...], w_ref[...], preferred_element_type=jnp.float32)
    out_ref[...] = dinv_ref[...] * xw


def _scale_mm(x, w, dinv):
    m, k = x.shape
    n = w.shape[1]
    return pl.pallas_call(
        _scale_mm_body,
        grid=(_G,),
        in_specs=[
            pl.BlockSpec((_BLK, k), lambda i: (i, 0)),
            pl.BlockSpec((k, n), lambda i: (0, 0)),
            pl.BlockSpec((_BLK, 1), lambda i: (i, 0)),
        ],
        out_specs=pl.BlockSpec((_BLK, n), lambda i: (i, 0)),
        out_shape=jax.ShapeDtypeStruct((m, n), jnp.float32),
    )(x, w, dinv)


def _layer_body(a_ref, y_ref, dinv_ref, b_ref, w_ref, out_ref):
    s = a_ref[0] + a_ref[1] + y_ref[...]
    h = jnp.maximum(dinv_ref[...] * s + b_ref[...], 0.0)
    out_ref[...] = dinv_ref[...] * jnp.dot(
        h, w_ref[...], preferred_element_type=jnp.float32)


def _layer(agg, y, dinv, b, w):
    d = y.shape[1]
    n = w.shape[1]
    return pl.pallas_call(
        _layer_body,
        grid=(_G,),
        in_specs=[
            pl.BlockSpec((NC, _BLK, d), lambda i: (0, i, 0)),
            pl.BlockSpec((_BLK, d), lambda i: (i, 0)),
            pl.BlockSpec((_BLK, 1), lambda i: (i, 0)),
            pl.BlockSpec((1, d), lambda i: (0, 0)),
            pl.BlockSpec((d, n), lambda i: (0, 0)),
        ],
        out_specs=pl.BlockSpec((_BLK, n), lambda i: (i, 0)),
        out_shape=jax.ShapeDtypeStruct((NP, n), jnp.float32),
    )(agg, y, dinv, b, w)


def _final_body(a_ref, y_ref, dinv_ref, b_ref, m1_ref, mb1_ref, m2_ref,
                mb2_ref, m3_ref, mb3_ref, out_ref):
    s = a_ref[0] + a_ref[1] + y_ref[...]
    h = jnp.maximum(dinv_ref[...] * s + b_ref[...], 0.0)
    h = jnp.maximum(
        jnp.dot(h, m1_ref[...], preferred_element_type=jnp.float32)
        + mb1_ref[...], 0.0)
    h = jnp.maximum(
        jnp.dot(h, m2_ref[...], preferred_element_type=jnp.float32)
        + mb2_ref[...], 0.0)
    out_ref[...] = jnp.dot(
        h, m3_ref[...], preferred_element_type=jnp.float32) + mb3_ref[...]


def _final(agg, y, dinv, b, m1, mb1, m2, mb2, m3, mb3):
    d = y.shape[1]
    nc = m3.shape[1]
    return pl.pallas_call(
        _final_body,
        grid=(_G,),
        in_specs=[
            pl.BlockSpec((NC, _BLK, d), lambda i: (0, i, 0)),
            pl.BlockSpec((_BLK, d), lambda i: (i, 0)),
            pl.BlockSpec((_BLK, 1), lambda i: (i, 0)),
            pl.BlockSpec((1, d), lambda i: (0, 0)),
            pl.BlockSpec((d, 64), lambda i: (0, 0)),
            pl.BlockSpec((1, 64), lambda i: (0, 0)),
            pl.BlockSpec((64, 64), lambda i: (0, 0)),
            pl.BlockSpec((1, 64), lambda i: (0, 0)),
            pl.BlockSpec((64, nc), lambda i: (0, 0)),
            pl.BlockSpec((1, nc), lambda i: (0, 0)),
        ],
        out_specs=pl.BlockSpec((_BLK, nc), lambda i: (i, 0)),
        out_shape=jax.ShapeDtypeStruct((NP, nc), jnp.float32),
    )(agg, y, dinv, b, m1, mb1, m2, mb2, m3, mb3)


def kernel(x, edge_index, W1, b1, W2, b2, W3, b3, W4, b4,
           M1, mb1, M2, mb2, M3, mb3):
    ei = edge_index.astype(jnp.int32)
    pad_iota = jnp.arange(EP - E, dtype=jnp.int32)
    srcf = jnp.concatenate([ei[0], pad_iota % N])
    dstf = jnp.concatenate([ei[1], N + pad_iota % (NP - N)])
    src = srcf.reshape(NW, NB, EB)
    dst = dstf.reshape(NW, NB, EB)

    xp = jnp.pad(x, ((0, NP - N), (0, 0)))
    W4p = jnp.pad(W4, ((0, 0), (0, 128 - W4.shape[1])))
    b4p = jnp.pad(b4, (0, 128 - b4.shape[0])).reshape(1, 128)
    M1p = jnp.pad(M1, ((0, 128 - M1.shape[0]), (0, 0)))
    mb3r = mb3.reshape(1, -1)
    b1r, b2r, b3r = b1.reshape(1, -1), b2.reshape(1, -1), b3.reshape(1, -1)
    mb1r, mb2r = mb1.reshape(1, -1), mb2.reshape(1, -1)

    _agg128 = _make_agg(128)
    zeros = jnp.zeros((NP, 128), jnp.float32)

    degp = _get_deg_kernel()(dst)
    dinv = _dinv_kernel(degp).reshape(NP, 1)

    y1 = _scale_mm(xp, W1, dinv)
    a1 = _agg128(y1, src, dst, zeros)
    y2 = _layer(a1, y1, dinv, b1r, W2)
    a2 = _agg128(y2, src, dst, zeros)
    y3 = _layer(a2, y2, dinv, b2r, W3)
    a3 = _agg128(y3, src, dst, zeros)
    y4 = _layer(a3, y3, dinv, b3r, W4p)
    a4 = _agg128(y4, src, dst, zeros)
    out = _final(a4, y4, dinv, b4p, M1p, mb1r, M2, mb2r, M3, mb3r)
    return out[:N]
